# baseline (speedup 1.0000x reference)
import functools
import math

import jax
import jax.numpy as jnp
from jax import lax
from jax.experimental import pallas as pl
from jax.experimental.pallas import tpu as pltpu
from jax.experimental.pallas import tpu_sc as plsc

D_MODEL = 64
SCALE = math.sqrt(D_MODEL)

NC = 2
NS = 16
NW = NC * NS

STEP = 128
LANES = 16


def _make_kernel(n_steps):
    mesh = plsc.VectorSubcoreMesh(core_axis_name="c", subcore_axis_name="s")

    @functools.partial(
        pl.kernel,
        mesh=mesh,
        compiler_params=pltpu.CompilerParams(needs_layout_passes=False),
        out_type=jax.ShapeDtypeStruct((NW, n_steps, STEP, D_MODEL), jnp.float32),
        scratch_types=[
            pltpu.VMEM((n_steps, STEP), jnp.int32),
            pltpu.VMEM((n_steps, STEP), jnp.int32),
            pltpu.VMEM((STEP, 2 * D_MODEL), jnp.float32),
            pltpu.VMEM((STEP, 2 * D_MODEL), jnp.float32),
            pltpu.VMEM((STEP, D_MODEL), jnp.float32),
            pltpu.VMEM((STEP, D_MODEL), jnp.float32),
            pltpu.SemaphoreType.DMA,
            pltpu.SemaphoreType.DMA,
        ],
    )
    def k(idx_hbm, par_hbm, tbl2_hbm, out_hbm,
          idx_v, par_v, rows0, rows1, outb0, outb1, gsem, ssem):
        wid = lax.axis_index("s") * NC + lax.axis_index("c")
        rows = (rows0, rows1)
        outb = (outb0, outb1)

        pltpu.sync_copy(idx_hbm.at[wid], idx_v)
        pltpu.sync_copy(par_hbm.at[wid], par_v)

        lanes = lax.iota(jnp.int32, LANES)

        def compute(j, rbuf, obuf):
            def group(gi, c2):
                rbase = gi * LANES
                row16 = lanes + rbase
                p16 = par_v[j, pl.ds(rbase, LANES)]
                colv = p16 * D_MODEL
                for c0 in range(D_MODEL):
                    v = plsc.load_gather(rbuf, [row16, colv + c0])
                    cvec = jnp.full((LANES,), c0, jnp.int32)
                    plsc.store_scatter(obuf, [row16, cvec], v * SCALE)
                return c2

            lax.fori_loop(0, STEP // LANES, group, 0)

        pltpu.async_copy(tbl2_hbm.at[idx_v.at[0]], rows0, gsem)

        def pair(i, carry):
            g0 = i * 2
            for b in range(2):
                g = g0 + b
                nb = 1 - b

                @pl.when(g + 1 < n_steps)
                def _():
                    pltpu.async_copy(
                        tbl2_hbm.at[idx_v.at[g + 1]], rows[nb], gsem)

                pltpu.make_async_copy(
                    tbl2_hbm.at[idx_v.at[g]], rows[b], gsem).wait()

                @pl.when(g >= 2)
                def _():
                    pltpu.make_async_copy(
                        outb[b], out_hbm.at[wid, g], ssem).wait()

                compute(g, rows[b], outb[b])
                pltpu.async_copy(outb[b], out_hbm.at[wid, g], ssem)
            return carry

        lax.fori_loop(0, n_steps // 2, pair, 0)

        pltpu.make_async_copy(outb0, out_hbm.at[wid, 0], ssem).wait()
        pltpu.make_async_copy(outb1, out_hbm.at[wid, 0], ssem).wait()

    return k


def kernel(x, table):
    b, s = x.shape
    total = b * s
    assert total % (NW * STEP) == 0 and (total // (NW * STEP)) % 2 == 0
    n_steps = total // (NW * STEP)
    v, d = table.shape
    tbl2 = table.reshape(v // 2, 2 * d)
    xf = x.reshape(-1).astype(jnp.int32)
    idx2 = (xf >> 1).reshape(NW, n_steps, STEP)
    par = (xf & 1).reshape(NW, n_steps, STEP)
    out = _make_kernel(n_steps)(idx2, par, tbl2)
    return out.reshape(b, s, D_MODEL)


# speedup vs baseline: 2.1617x; 2.1617x over previous
import functools
import math

import jax
import jax.numpy as jnp
from jax import lax
from jax.experimental import pallas as pl
from jax.experimental.pallas import tpu as pltpu
from jax.experimental.pallas import tpu_sc as plsc

D_MODEL = 64
SCALE = math.sqrt(D_MODEL)

NC = 2
NS = 16
NW = NC * NS

STEP = 128
LANES = 16


def _make_kernel(n_steps):
    mesh = plsc.VectorSubcoreMesh(core_axis_name="c", subcore_axis_name="s")

    @functools.partial(
        pl.kernel,
        mesh=mesh,
        compiler_params=pltpu.CompilerParams(needs_layout_passes=False),
        out_type=jax.ShapeDtypeStruct((NW, n_steps, STEP, D_MODEL), jnp.float32),
        scratch_types=[
            pltpu.VMEM((n_steps, STEP), jnp.int32),
            pltpu.VMEM((n_steps, STEP), jnp.int32),
            pltpu.VMEM((STEP, 2 * D_MODEL), jnp.float32),
            pltpu.VMEM((STEP, 2 * D_MODEL), jnp.float32),
            pltpu.VMEM((STEP, D_MODEL), jnp.float32),
            pltpu.VMEM((STEP, D_MODEL), jnp.float32),
            pltpu.SemaphoreType.DMA,
            pltpu.SemaphoreType.DMA,
        ],
    )
    def k(idx_hbm, off_hbm, tbl2_hbm, out_hbm,
          idx_v, off_v, rows0, rows1, outb0, outb1, gsem, ssem):
        wid = lax.axis_index("s") * NC + lax.axis_index("c")
        rows = (rows0, rows1)
        outb = (outb0, outb1)

        pltpu.sync_copy(idx_hbm.at[wid], idx_v)
        pltpu.sync_copy(off_hbm.at[wid], off_v)

        def compute(j, rbuf, obuf):
            def group(gi, c2):
                rbase = gi * LANES
                ov = off_v[j, pl.ds(rbase, LANES)]
                for k in range(LANES):
                    off = ov[k]
                    r = rbase + k
                    for cc in range(D_MODEL // LANES):
                        src = pl.ds(off + cc * LANES, LANES)
                        obuf[r, pl.ds(cc * LANES, LANES)] = rbuf[r, src] * SCALE
                return c2

            lax.fori_loop(0, STEP // LANES, group, 0)

        pltpu.async_copy(tbl2_hbm.at[idx_v.at[0]], rows0, gsem)

        def pair(i, carry):
            g0 = i * 2
            for b in range(2):
                g = g0 + b
                nb = 1 - b

                @pl.when(g + 1 < n_steps)
                def _():
                    pltpu.async_copy(
                        tbl2_hbm.at[idx_v.at[g + 1]], rows[nb], gsem)

                pltpu.make_async_copy(
                    tbl2_hbm.at[idx_v.at[g]], rows[b], gsem).wait()

                @pl.when(g >= 2)
                def _():
                    pltpu.make_async_copy(
                        outb[b], out_hbm.at[wid, g], ssem).wait()

                compute(g, rows[b], outb[b])
                pltpu.async_copy(outb[b], out_hbm.at[wid, g], ssem)
            return carry

        lax.fori_loop(0, n_steps // 2, pair, 0)

        pltpu.make_async_copy(outb0, out_hbm.at[wid, 0], ssem).wait()
        pltpu.make_async_copy(outb1, out_hbm.at[wid, 0], ssem).wait()

    return k


def kernel(x, table):
    b, s = x.shape
    total = b * s
    assert total % (NW * STEP) == 0 and (total // (NW * STEP)) % 2 == 0
    n_steps = total // (NW * STEP)
    v, d = table.shape
    tbl2 = table.reshape(v // 2, 2 * d)
    xf = x.reshape(-1).astype(jnp.int32)
    idx2 = (xf >> 1).reshape(NW, n_steps, STEP)
    off = ((xf & 1) * D_MODEL).reshape(NW, n_steps, STEP)
    out = _make_kernel(n_steps)(idx2, off, tbl2)
    return out.reshape(b, s, D_MODEL)
